# baseline (speedup 1.0000x reference)
import jax
import jax.numpy as jnp
from jax.experimental import pallas as pl
from jax.experimental.pallas import tpu as pltpu

EMBED = 512
HIDDEN = 512
BATCH = 64
SEQ = 32
G4 = 4 * HIDDEN
GC = 2 * G4
TS = 4


def _dotT(a, w):
    return jax.lax.dot_general(
        a, w, (((1,), (1,)), ((), ())), preferred_element_type=jnp.float32)


def _xgates_kernel(x_ref, w_ref, b_ref, out_ref):
    x = x_ref[...].reshape(TS * BATCH, EMBED).astype(jnp.bfloat16)
    acc = _dotT(x, w_ref[...]) + b_ref[...]
    out_ref[...] = acc.reshape(TS, BATCH, GC)


def _scan_kernel(tok_ref, xg_ref, wh_ref, out_ref, hF_ref, cF_ref, h_scr, c_scr):
    t = pl.program_id(0)

    @pl.when(t == 0)
    def _init():
        h_scr[...] = jnp.zeros_like(h_scr)
        c_scr[...] = jnp.zeros_like(c_scr)

    h = h_scr[...]
    c = c_scr[...]
    g = xg_ref[0] + _dotT(h.astype(jnp.bfloat16), wh_ref[...])

    i0 = jax.nn.sigmoid(g[:, 0 * HIDDEN:1 * HIDDEN])
    f0 = jax.nn.sigmoid(g[:, 1 * HIDDEN:2 * HIDDEN])
    g0 = jnp.tanh(g[:, 2 * HIDDEN:3 * HIDDEN])
    o0 = jax.nn.sigmoid(g[:, 3 * HIDDEN:4 * HIDDEN])
    i1 = jax.nn.sigmoid(g[:, 4 * HIDDEN:5 * HIDDEN])
    f1 = jax.nn.sigmoid(g[:, 5 * HIDDEN:6 * HIDDEN])
    g1 = jnp.tanh(g[:, 6 * HIDDEN:7 * HIDDEN])
    o1 = jax.nn.sigmoid(g[:, 7 * HIDDEN:8 * HIDDEN])

    cA = f0 * c + i0 * g0
    hA = o0 * jnp.tanh(cA)
    cB = f1 * c + i1 * g1
    hB = o1 * jnp.tanh(cB)

    m = (tok_ref[0] % 2) == 1
    h_new = jnp.where(m, hB, hA)
    c_new = jnp.where(m, cB, cA)

    h_scr[...] = h_new
    c_scr[...] = c_new
    out_ref[0] = h_new
    hF_ref[...] = h_new
    cF_ref[...] = c_new


def kernel(input, input_embed, W_ih0, W_hh0, b_ih0, b_hh0, W_ih1, W_hh1, b_ih1, b_hh1):
    Wx = jnp.concatenate([W_ih0, W_ih1], axis=0).astype(jnp.bfloat16)
    Wh = jnp.concatenate([W_hh0, W_hh1], axis=0).astype(jnp.bfloat16)
    b = jnp.concatenate([b_ih0 + b_hh0, b_ih1 + b_hh1]).reshape(1, GC)
    tok = input.T.reshape(SEQ, BATCH, 1)

    xg = pl.pallas_call(
        _xgates_kernel,
        grid=(SEQ // TS,),
        in_specs=[
            pl.BlockSpec((TS, BATCH, EMBED), lambda i: (i, 0, 0)),
            pl.BlockSpec((GC, EMBED), lambda i: (0, 0)),
            pl.BlockSpec((1, GC), lambda i: (0, 0)),
        ],
        out_specs=pl.BlockSpec((TS, BATCH, GC), lambda i: (i, 0, 0)),
        out_shape=jax.ShapeDtypeStruct((SEQ, BATCH, GC), jnp.float32),
    )(input_embed, Wx, b)

    out, hF, cF = pl.pallas_call(
        _scan_kernel,
        grid=(SEQ,),
        in_specs=[
            pl.BlockSpec((1, BATCH, 1), lambda t: (t, 0, 0)),
            pl.BlockSpec((1, BATCH, GC), lambda t: (t, 0, 0)),
            pl.BlockSpec((GC, HIDDEN), lambda t: (0, 0)),
        ],
        out_specs=[
            pl.BlockSpec((1, BATCH, HIDDEN), lambda t: (t, 0, 0)),
            pl.BlockSpec((BATCH, HIDDEN), lambda t: (0, 0)),
            pl.BlockSpec((BATCH, HIDDEN), lambda t: (0, 0)),
        ],
        out_shape=[
            jax.ShapeDtypeStruct((SEQ, BATCH, HIDDEN), jnp.float32),
            jax.ShapeDtypeStruct((BATCH, HIDDEN), jnp.float32),
            jax.ShapeDtypeStruct((BATCH, HIDDEN), jnp.float32),
        ],
        scratch_shapes=[
            pltpu.VMEM((BATCH, HIDDEN), jnp.float32),
            pltpu.VMEM((BATCH, HIDDEN), jnp.float32),
        ],
    )(tok, xg, Wh)

    return out, (hF, cF)


# speedup vs baseline: 1.1034x; 1.1034x over previous
import jax
import jax.numpy as jnp
from jax.experimental import pallas as pl
from jax.experimental.pallas import tpu as pltpu

EMBED = 512
HIDDEN = 512
BATCH = 64
SEQ = 32
G4 = 4 * HIDDEN
GC = 2 * G4
TS = 8


def _dotT(a, w):
    return jax.lax.dot_general(
        a, w, (((1,), (1,)), ((), ())), preferred_element_type=jnp.float32)


def _fused_kernel(tok_ref, x_ref, wx_ref, bx_ref, wh_ref,
                  out_ref, hF_ref, cF_ref,
                  xg_scr, h_scr, c_scr):
    t = pl.program_id(0)

    @pl.when(t == 0)
    def _init():
        h_scr[...] = jnp.zeros_like(h_scr)
        c_scr[...] = jnp.zeros_like(c_scr)

    @pl.when(t % TS == 0)
    def _xgates():
        x = x_ref[...].reshape(TS * BATCH, EMBED).astype(jnp.bfloat16)
        acc = _dotT(x, wx_ref[...]) + bx_ref[...]
        xg_scr[...] = acc.reshape(TS, BATCH, GC)

    h = h_scr[...]
    c = c_scr[...]
    g = xg_scr[t % TS] + _dotT(h.astype(jnp.bfloat16), wh_ref[...])

    i0 = jax.nn.sigmoid(g[:, 0 * HIDDEN:1 * HIDDEN])
    f0 = jax.nn.sigmoid(g[:, 1 * HIDDEN:2 * HIDDEN])
    g0 = jnp.tanh(g[:, 2 * HIDDEN:3 * HIDDEN])
    o0 = jax.nn.sigmoid(g[:, 3 * HIDDEN:4 * HIDDEN])
    i1 = jax.nn.sigmoid(g[:, 4 * HIDDEN:5 * HIDDEN])
    f1 = jax.nn.sigmoid(g[:, 5 * HIDDEN:6 * HIDDEN])
    g1 = jnp.tanh(g[:, 6 * HIDDEN:7 * HIDDEN])
    o1 = jax.nn.sigmoid(g[:, 7 * HIDDEN:8 * HIDDEN])

    cA = f0 * c + i0 * g0
    hA = o0 * jnp.tanh(cA)
    cB = f1 * c + i1 * g1
    hB = o1 * jnp.tanh(cB)

    m = (tok_ref[0] % 2) == 1
    h_new = jnp.where(m, hB, hA)
    c_new = jnp.where(m, cB, cA)

    h_scr[...] = h_new
    c_scr[...] = c_new
    out_ref[0] = h_new
    hF_ref[...] = h_new
    cF_ref[...] = c_new


def kernel(input, input_embed, W_ih0, W_hh0, b_ih0, b_hh0, W_ih1, W_hh1, b_ih1, b_hh1):
    Wx = jnp.concatenate([W_ih0, W_ih1], axis=0).astype(jnp.bfloat16)
    Wh = jnp.concatenate([W_hh0, W_hh1], axis=0).astype(jnp.bfloat16)
    b = jnp.concatenate([b_ih0 + b_hh0, b_ih1 + b_hh1]).reshape(1, GC)
    tok = input.T.reshape(SEQ, BATCH, 1)

    out, hF, cF = pl.pallas_call(
        _fused_kernel,
        grid=(SEQ,),
        in_specs=[
            pl.BlockSpec((1, BATCH, 1), lambda t: (t, 0, 0)),
            pl.BlockSpec((TS, BATCH, EMBED), lambda t: (t // TS, 0, 0)),
            pl.BlockSpec((GC, EMBED), lambda t: (0, 0)),
            pl.BlockSpec((1, GC), lambda t: (0, 0)),
            pl.BlockSpec((GC, HIDDEN), lambda t: (0, 0)),
        ],
        out_specs=[
            pl.BlockSpec((1, BATCH, HIDDEN), lambda t: (t, 0, 0)),
            pl.BlockSpec((BATCH, HIDDEN), lambda t: (0, 0)),
            pl.BlockSpec((BATCH, HIDDEN), lambda t: (0, 0)),
        ],
        out_shape=[
            jax.ShapeDtypeStruct((SEQ, BATCH, HIDDEN), jnp.float32),
            jax.ShapeDtypeStruct((BATCH, HIDDEN), jnp.float32),
            jax.ShapeDtypeStruct((BATCH, HIDDEN), jnp.float32),
        ],
        scratch_shapes=[
            pltpu.VMEM((TS, BATCH, GC), jnp.float32),
            pltpu.VMEM((BATCH, HIDDEN), jnp.float32),
            pltpu.VMEM((BATCH, HIDDEN), jnp.float32),
        ],
    )(tok, input_embed, Wx, b, Wh)

    return out, (hF, cF)


# speedup vs baseline: 1.2353x; 1.1196x over previous
import jax
import jax.numpy as jnp
from jax.experimental import pallas as pl
from jax.experimental.pallas import tpu as pltpu

EMBED = 512
HIDDEN = 512
BATCH = 64
SEQ = 32
G4 = 4 * HIDDEN
GC = 2 * G4
TS = 8


def _dotT(a, w):
    return jax.lax.dot_general(
        a, w, (((1,), (1,)), ((), ())), preferred_element_type=jnp.float32)


def _fused_kernel(tok_ref, x_ref, wih0_ref, wih1_ref, whh0_ref, whh1_ref,
                  bi0_ref, bh0_ref, bi1_ref, bh1_ref,
                  out_ref, hF_ref, cF_ref,
                  wx_scr, wh_scr, bx_scr, xg_scr, h_scr, c_scr):
    t = pl.program_id(0)

    @pl.when(t == 0)
    def _prep():
        h_scr[...] = jnp.zeros_like(h_scr)
        c_scr[...] = jnp.zeros_like(c_scr)
        wx_scr[:G4] = wih0_ref[...].astype(jnp.bfloat16)
        wx_scr[G4:] = wih1_ref[...].astype(jnp.bfloat16)
        wh_scr[:G4] = whh0_ref[...].astype(jnp.bfloat16)
        wh_scr[G4:] = whh1_ref[...].astype(jnp.bfloat16)
        bx_scr[:, :G4] = bi0_ref[...] + bh0_ref[...]
        bx_scr[:, G4:] = bi1_ref[...] + bh1_ref[...]

    @pl.when(t % TS == 0)
    def _xgates():
        x = x_ref[...].reshape(TS * BATCH, EMBED).astype(jnp.bfloat16)
        acc = _dotT(x, wx_scr[...]) + bx_scr[...]
        xg_scr[...] = acc.reshape(TS, BATCH, GC)

    h = h_scr[...]
    c = c_scr[...]
    g = xg_scr[t % TS] + _dotT(h.astype(jnp.bfloat16), wh_scr[...])

    m = (tok_ref[0] % 2) == 1
    gi = jnp.where(m, g[:, 4 * HIDDEN:5 * HIDDEN], g[:, 0 * HIDDEN:1 * HIDDEN])
    gf = jnp.where(m, g[:, 5 * HIDDEN:6 * HIDDEN], g[:, 1 * HIDDEN:2 * HIDDEN])
    gg = jnp.where(m, g[:, 6 * HIDDEN:7 * HIDDEN], g[:, 2 * HIDDEN:3 * HIDDEN])
    go = jnp.where(m, g[:, 7 * HIDDEN:8 * HIDDEN], g[:, 3 * HIDDEN:4 * HIDDEN])

    c_new = jax.nn.sigmoid(gf) * c + jax.nn.sigmoid(gi) * jnp.tanh(gg)
    h_new = jax.nn.sigmoid(go) * jnp.tanh(c_new)

    h_scr[...] = h_new
    c_scr[...] = c_new
    out_ref[0] = h_new
    hF_ref[...] = h_new
    cF_ref[...] = c_new


def kernel(input, input_embed, W_ih0, W_hh0, b_ih0, b_hh0, W_ih1, W_hh1, b_ih1, b_hh1):
    tok = input.T.reshape(SEQ, BATCH, 1)
    resident = lambda shape: pl.BlockSpec(shape, lambda t: tuple(0 for _ in shape))

    out, hF, cF = pl.pallas_call(
        _fused_kernel,
        grid=(SEQ,),
        in_specs=[
            pl.BlockSpec((1, BATCH, 1), lambda t: (t, 0, 0)),
            pl.BlockSpec((TS, BATCH, EMBED), lambda t: (t // TS, 0, 0)),
            resident((G4, EMBED)),
            resident((G4, EMBED)),
            resident((G4, HIDDEN)),
            resident((G4, HIDDEN)),
            resident((1, G4)),
            resident((1, G4)),
            resident((1, G4)),
            resident((1, G4)),
        ],
        out_specs=[
            pl.BlockSpec((1, BATCH, HIDDEN), lambda t: (t, 0, 0)),
            resident((BATCH, HIDDEN)),
            resident((BATCH, HIDDEN)),
        ],
        out_shape=[
            jax.ShapeDtypeStruct((SEQ, BATCH, HIDDEN), jnp.float32),
            jax.ShapeDtypeStruct((BATCH, HIDDEN), jnp.float32),
            jax.ShapeDtypeStruct((BATCH, HIDDEN), jnp.float32),
        ],
        scratch_shapes=[
            pltpu.VMEM((GC, EMBED), jnp.bfloat16),
            pltpu.VMEM((GC, HIDDEN), jnp.bfloat16),
            pltpu.VMEM((1, GC), jnp.float32),
            pltpu.VMEM((TS, BATCH, GC), jnp.float32),
            pltpu.VMEM((BATCH, HIDDEN), jnp.float32),
            pltpu.VMEM((BATCH, HIDDEN), jnp.float32),
        ],
    )(tok, input_embed, W_ih0, W_ih1, W_hh0, W_hh1,
      b_ih0.reshape(1, G4), b_hh0.reshape(1, G4),
      b_ih1.reshape(1, G4), b_hh1.reshape(1, G4))

    return out, (hF, cF)


# speedup vs baseline: 1.3186x; 1.0674x over previous
import jax
import jax.numpy as jnp
from jax.experimental import pallas as pl
from jax.experimental.pallas import tpu as pltpu

EMBED = 512
HIDDEN = 512
BATCH = 64
SEQ = 32
G4 = 4 * HIDDEN
GC = 2 * G4
TS = 8
UNROLL = 4


def _dotT(a, w):
    return jax.lax.dot_general(
        a, w, (((1,), (1,)), ((), ())), preferred_element_type=jnp.float32)


def _fused_kernel(tok_ref, x_ref, wih0_ref, wih1_ref, whh0_ref, whh1_ref,
                  bi0_ref, bh0_ref, bi1_ref, bh1_ref,
                  out_ref, hF_ref, cF_ref,
                  wx_scr, wh_scr, bx_scr, xg_scr, h_scr, c_scr):
    i = pl.program_id(0)

    @pl.when(i == 0)
    def _prep():
        h_scr[...] = jnp.zeros_like(h_scr)
        c_scr[...] = jnp.zeros_like(c_scr)
        wx_scr[:G4] = wih0_ref[...].astype(jnp.bfloat16)
        wx_scr[G4:] = wih1_ref[...].astype(jnp.bfloat16)
        wh_scr[:G4] = whh0_ref[...].astype(jnp.bfloat16)
        wh_scr[G4:] = whh1_ref[...].astype(jnp.bfloat16)
        bx_scr[:, :G4] = bi0_ref[...] + bh0_ref[...]
        bx_scr[:, G4:] = bi1_ref[...] + bh1_ref[...]

    @pl.when(i % (TS // UNROLL) == 0)
    def _xgates():
        x = x_ref[...].reshape(TS * BATCH, EMBED).astype(jnp.bfloat16)
        acc = _dotT(x, wx_scr[...]) + bx_scr[...]
        xg_scr[...] = acc.reshape(TS, BATCH, GC)

    base = (i % (TS // UNROLL)) * UNROLL
    h = h_scr[...]
    c = c_scr[...]
    for k in range(UNROLL):
        g = xg_scr[base + k] + _dotT(h.astype(jnp.bfloat16), wh_scr[...])

        m = (tok_ref[i * UNROLL + k] % 2) == 1
        gi = jnp.where(m, g[:, 4 * HIDDEN:5 * HIDDEN], g[:, 0 * HIDDEN:1 * HIDDEN])
        gf = jnp.where(m, g[:, 5 * HIDDEN:6 * HIDDEN], g[:, 1 * HIDDEN:2 * HIDDEN])
        gg = jnp.where(m, g[:, 6 * HIDDEN:7 * HIDDEN], g[:, 2 * HIDDEN:3 * HIDDEN])
        go = jnp.where(m, g[:, 7 * HIDDEN:8 * HIDDEN], g[:, 3 * HIDDEN:4 * HIDDEN])

        c = jax.nn.sigmoid(gf) * c + jax.nn.sigmoid(gi) * jnp.tanh(gg)
        h = jax.nn.sigmoid(go) * jnp.tanh(c)
        out_ref[k] = h

    h_scr[...] = h
    c_scr[...] = c
    hF_ref[...] = h
    cF_ref[...] = c


def kernel(input, input_embed, W_ih0, W_hh0, b_ih0, b_hh0, W_ih1, W_hh1, b_ih1, b_hh1):
    tok = input.T.reshape(SEQ, BATCH, 1)
    resident = lambda shape: pl.BlockSpec(shape, lambda t: tuple(0 for _ in shape))

    out, hF, cF = pl.pallas_call(
        _fused_kernel,
        grid=(SEQ // UNROLL,),
        in_specs=[
            resident((SEQ, BATCH, 1)),
            pl.BlockSpec((TS, BATCH, EMBED), lambda i: (i // (TS // UNROLL), 0, 0)),
            resident((G4, EMBED)),
            resident((G4, EMBED)),
            resident((G4, HIDDEN)),
            resident((G4, HIDDEN)),
            resident((1, G4)),
            resident((1, G4)),
            resident((1, G4)),
            resident((1, G4)),
        ],
        out_specs=[
            pl.BlockSpec((UNROLL, BATCH, HIDDEN), lambda i: (i, 0, 0)),
            resident((BATCH, HIDDEN)),
            resident((BATCH, HIDDEN)),
        ],
        out_shape=[
            jax.ShapeDtypeStruct((SEQ, BATCH, HIDDEN), jnp.float32),
            jax.ShapeDtypeStruct((BATCH, HIDDEN), jnp.float32),
            jax.ShapeDtypeStruct((BATCH, HIDDEN), jnp.float32),
        ],
        scratch_shapes=[
            pltpu.VMEM((GC, EMBED), jnp.bfloat16),
            pltpu.VMEM((GC, HIDDEN), jnp.bfloat16),
            pltpu.VMEM((1, GC), jnp.float32),
            pltpu.VMEM((TS, BATCH, GC), jnp.float32),
            pltpu.VMEM((BATCH, HIDDEN), jnp.float32),
            pltpu.VMEM((BATCH, HIDDEN), jnp.float32),
        ],
    )(tok, input_embed, W_ih0, W_ih1, W_hh0, W_hh1,
      b_ih0.reshape(1, G4), b_hh0.reshape(1, G4),
      b_ih1.reshape(1, G4), b_hh1.reshape(1, G4))

    return out, (hF, cF)


# speedup vs baseline: 1.3646x; 1.0349x over previous
import jax
import jax.numpy as jnp
from jax.experimental import pallas as pl
from jax.experimental.pallas import tpu as pltpu

EMBED = 512
HIDDEN = 512
BATCH = 64
SEQ = 32
G4 = 4 * HIDDEN
GC = 2 * G4
UNROLL = 8


def _dotT(a, w):
    return jax.lax.dot_general(
        a, w, (((1,), (1,)), ((), ())), preferred_element_type=jnp.float32)


def _fused_kernel(tok_ref, x_ref, wih0_ref, wih1_ref, whh0_ref, whh1_ref,
                  bi0_ref, bh0_ref, bi1_ref, bh1_ref,
                  out_ref, hF_ref, cF_ref,
                  wx_scr, wh_scr, bx_scr, par_scr, xg_scr, h_scr, c_scr):
    i = pl.program_id(0)

    @pl.when(i == 0)
    def _prep():
        h_scr[...] = jnp.zeros_like(h_scr)
        c_scr[...] = jnp.zeros_like(c_scr)
        wx_scr[:G4] = wih0_ref[...].astype(jnp.bfloat16)
        wx_scr[G4:] = wih1_ref[...].astype(jnp.bfloat16)
        wh_scr[:G4] = whh0_ref[...].astype(jnp.bfloat16)
        wh_scr[G4:] = whh1_ref[...].astype(jnp.bfloat16)
        bx_scr[:, :G4] = bi0_ref[...] + bh0_ref[...]
        bx_scr[:, G4:] = bi1_ref[...] + bh1_ref[...]
        par_scr[...] = (tok_ref[...] % 2).astype(jnp.float32)

    x = x_ref[...].reshape(UNROLL * BATCH, EMBED).astype(jnp.bfloat16)
    xg_scr[...] = (_dotT(x, wx_scr[...]) + bx_scr[...]).reshape(UNROLL, BATCH, GC)

    lane = jax.lax.broadcasted_iota(jnp.int32, (BATCH, SEQ), 1)
    h = h_scr[...]
    c = c_scr[...]
    for k in range(UNROLL):
        g = xg_scr[k] + _dotT(h.astype(jnp.bfloat16), wh_scr[...])

        t = i * UNROLL + k
        mcol = jnp.sum(jnp.where(lane == t, par_scr[...], 0.0),
                       axis=1, keepdims=True)
        m = mcol > 0.5
        gi = jnp.where(m, g[:, 4 * HIDDEN:5 * HIDDEN], g[:, 0 * HIDDEN:1 * HIDDEN])
        gf = jnp.where(m, g[:, 5 * HIDDEN:6 * HIDDEN], g[:, 1 * HIDDEN:2 * HIDDEN])
        gg = jnp.where(m, g[:, 6 * HIDDEN:7 * HIDDEN], g[:, 2 * HIDDEN:3 * HIDDEN])
        go = jnp.where(m, g[:, 7 * HIDDEN:8 * HIDDEN], g[:, 3 * HIDDEN:4 * HIDDEN])

        c = jax.nn.sigmoid(gf) * c + jax.nn.sigmoid(gi) * jnp.tanh(gg)
        h = jax.nn.sigmoid(go) * jnp.tanh(c)
        out_ref[k] = h

    h_scr[...] = h
    c_scr[...] = c
    hF_ref[...] = h
    cF_ref[...] = c


def kernel(input, input_embed, W_ih0, W_hh0, b_ih0, b_hh0, W_ih1, W_hh1, b_ih1, b_hh1):
    resident = lambda shape: pl.BlockSpec(shape, lambda t: tuple(0 for _ in shape))

    out, hF, cF = pl.pallas_call(
        _fused_kernel,
        grid=(SEQ // UNROLL,),
        in_specs=[
            resident((BATCH, SEQ)),
            pl.BlockSpec((UNROLL, BATCH, EMBED), lambda i: (i, 0, 0)),
            resident((G4, EMBED)),
            resident((G4, EMBED)),
            resident((G4, HIDDEN)),
            resident((G4, HIDDEN)),
            resident((1, G4)),
            resident((1, G4)),
            resident((1, G4)),
            resident((1, G4)),
        ],
        out_specs=[
            pl.BlockSpec((UNROLL, BATCH, HIDDEN), lambda i: (i, 0, 0)),
            resident((BATCH, HIDDEN)),
            resident((BATCH, HIDDEN)),
        ],
        out_shape=[
            jax.ShapeDtypeStruct((SEQ, BATCH, HIDDEN), jnp.float32),
            jax.ShapeDtypeStruct((BATCH, HIDDEN), jnp.float32),
            jax.ShapeDtypeStruct((BATCH, HIDDEN), jnp.float32),
        ],
        scratch_shapes=[
            pltpu.VMEM((GC, EMBED), jnp.bfloat16),
            pltpu.VMEM((GC, HIDDEN), jnp.bfloat16),
            pltpu.VMEM((1, GC), jnp.float32),
            pltpu.VMEM((BATCH, SEQ), jnp.float32),
            pltpu.VMEM((UNROLL, BATCH, GC), jnp.float32),
            pltpu.VMEM((BATCH, HIDDEN), jnp.float32),
            pltpu.VMEM((BATCH, HIDDEN), jnp.float32),
        ],
    )(input, input_embed, W_ih0, W_ih1, W_hh0, W_hh1,
      b_ih0.reshape(1, G4), b_hh0.reshape(1, G4),
      b_ih1.reshape(1, G4), b_hh1.reshape(1, G4))

    return out, (hF, cF)
